# baseline (speedup 1.0000x reference)
import functools

import jax
import jax.numpy as jnp
from jax import lax
from jax.experimental import pallas as pl
from jax.experimental.pallas import tpu as pltpu
from jax.experimental.pallas import tpu_sc as plsc

N = 50000
D = 256
D2 = D // 2
S = 5
L = 16

NC = 2
NS = 16
NW = NC * NS

NPW0 = 1600
NPW1 = 1600
CAP = NW // 2 * (NPW0 + NPW1)
CHUNK = 40

BM = 5000
GRID = N // BM


def _sc_gather_max_body(table_hbm, idx_hbm, out_hbm, idx_v,
                        rows0, rows1, out0, out1,
                        gsem0, gsem1, osem0, osem1):
    c = lax.axis_index("c")
    s_ax = lax.axis_index("s")
    node_base = jnp.where(c == 0, s_ax * NPW0, NS * NPW0 + s_ax * NPW1)
    cap = jnp.where(c == 0, NPW0, NPW1)
    cnt = jnp.clip(N - node_base, 0, cap)
    nchunk = cnt // CHUNK
    npair = nchunk // 2

    @pl.when(c == 0)
    def _():
        pltpu.sync_copy(idx_hbm.at[pl.ds(node_base * S, NPW0 * S)],
                        idx_v.at[pl.ds(0, NPW0 * S)])

    @pl.when(c != 0)
    def _():
        pltpu.sync_copy(idx_hbm.at[pl.ds(node_base * S, NPW1 * S)],
                        idx_v.at[pl.ds(0, NPW1 * S)])

    rows = (rows0, rows1)
    outs = (out0, out1)
    gsems = (gsem0, gsem1)
    osems = (osem0, osem1)

    H0 = 96
    H1 = CHUNK * S - H0

    def gather_start(ci, b):
        base = ci * (CHUNK * S)
        pltpu.make_async_copy(
            table_hbm.at[idx_v.at[pl.ds(base, H0)]],
            rows[b].at[pl.ds(0, H0)], gsems[b]).start()
        pltpu.make_async_copy(
            table_hbm.at[idx_v.at[pl.ds(base + H0, H1)]],
            rows[b].at[pl.ds(H0, H1)], gsems[b]).start()

    def gather_wait(ci, b):
        pltpu.make_async_copy(
            table_hbm.at[idx_v.at[pl.ds(ci * (CHUNK * S), CHUNK * S)]],
            rows[b], gsems[b]).wait()

    def writeback(ci, b):
        return pltpu.make_async_copy(
            outs[b], out_hbm.at[pl.ds(node_base + ci * CHUNK, CHUNK)],
            osems[b])

    @pl.when(npair > 0)
    def _():
        gather_start(0, 0)

    def pair_body(pi, carry):
        for b in range(2):
            ci = pi * 2 + b
            nb = 1 - b

            @pl.when(ci + 1 < nchunk)
            def _():
                gather_start(ci + 1, nb)

            gather_wait(ci, b)

            @pl.when(ci >= 2)
            def _():
                writeback(ci - 2, b).wait()

            @plsc.parallel_loop(0, CHUNK, step=1, unroll=2)
            def node_body(j):
                r = j * S
                for k in range(D2 // L):
                    sl = pl.ds(k * L, L)
                    w = rows[b][r, sl]
                    hi = w
                    lo = w << 16
                    for s in range(1, S):
                        w = rows[b][r + s, sl]
                        hi = jnp.maximum(hi, w)
                        lo = jnp.maximum(lo, w << 16)
                    outs[b][j, sl] = (hi & -65536) | lax.shift_right_logical(lo, 16)
            writeback(ci, b).start()
        return carry

    lax.fori_loop(0, npair, pair_body, 0, unroll=False)

    @pl.when(npair > 0)
    def _():
        writeback(nchunk - 2, 0).wait()
        writeback(nchunk - 1, 1).wait()


def _sc_gather_max(table, idx_flat):
    mesh = plsc.VectorSubcoreMesh(core_axis_name="c", subcore_axis_name="s")
    return pl.kernel(
        _sc_gather_max_body,
        out_type=jax.ShapeDtypeStruct((N, D2), jnp.int32),
        mesh=mesh,
        scratch_types=[
            pltpu.VMEM((NPW0 * S,), jnp.int32),
            pltpu.VMEM((CHUNK * S, D2), jnp.int32),
            pltpu.VMEM((CHUNK * S, D2), jnp.int32),
            pltpu.VMEM((CHUNK, D2), jnp.int32),
            pltpu.VMEM((CHUNK, D2), jnp.int32),
            pltpu.SemaphoreType.DMA,
            pltpu.SemaphoreType.DMA,
            pltpu.SemaphoreType.DMA,
            pltpu.SemaphoreType.DMA,
        ],
        name="sc_gather_max",
    )(table, idx_flat)


def _pack_bf16(x):
    u = jax.lax.bitcast_convert_type(x[:, :D2], jnp.uint32)
    v = jax.lax.bitcast_convert_type(x[:, D2:], jnp.uint32)
    lo = (u + 0x7FFF + ((u >> 16) & 1)) >> 16
    hi = (v + 0x7FFF + ((v >> 16) & 1)) & jnp.uint32(0xFFFF0000)
    return jax.lax.bitcast_convert_type(lo | hi, jnp.int32)


def _unpack_bf16(p):
    u = jax.lax.bitcast_convert_type(p, jnp.uint32)
    lo = jax.lax.bitcast_convert_type(u << 16, jnp.float32)
    hi = jax.lax.bitcast_convert_type(u & jnp.uint32(0xFFFF0000), jnp.float32)
    return jnp.concatenate([lo, hi], axis=1)


def _mm_relu_body(x_ref, w_ref, b_ref, o_ref):
    acc = jnp.dot(x_ref[...], w_ref[...], preferred_element_type=jnp.float32)
    o_ref[...] = _pack_bf16(jnp.maximum(acc + b_ref[...], 0.0))


def _lin_body(x_ref, w_ref, b_ref, o_ref):
    acc = jnp.dot(x_ref[...], w_ref[...], preferred_element_type=jnp.float32)
    o_ref[...] = acc + b_ref[...]


def _fc0_body(ft_ref, agg_ref, wb_ref, z_ref, s_ref, ss_ref):
    z = ft_ref[...] + jnp.dot(_unpack_bf16(agg_ref[...]), wb_ref[...],
                              preferred_element_type=jnp.float32)
    z = jnp.maximum(z, 0.0)
    z_ref[...] = z

    @pl.when(pl.program_id(0) == 0)
    def _():
        s_ref[...] = jnp.zeros_like(s_ref)
        ss_ref[...] = jnp.zeros_like(ss_ref)

    s_ref[...] += jnp.sum(z, axis=0, keepdims=True)
    ss_ref[...] += jnp.sum(z * z, axis=0, keepdims=True)


def _bn_body(z_ref, s_ref, ss_ref, g_ref, be_ref, wp_ref, bp_ref,
             on_ref, h1_ref):
    mean = s_ref[...] / N
    var = ss_ref[...] / N - mean * mean
    y = (z_ref[...] - mean) / jnp.sqrt(var + 1e-5) * g_ref[...] + be_ref[...]
    nrm = jnp.sqrt(jnp.sum(y * y, axis=1, keepdims=True)) + 1e-6
    on = y / nrm
    on_ref[...] = on
    acc = jnp.dot(on, wp_ref[...], preferred_element_type=jnp.float32)
    h1_ref[...] = _pack_bf16(jnp.maximum(acc + bp_ref[...], 0.0))


def _fc1_body(g_ref, agg_ref, wb_ref, o_ref):
    o_ref[...] = g_ref[...] + jnp.dot(_unpack_bf16(agg_ref[...]), wb_ref[...],
                                      preferred_element_type=jnp.float32)


def _row_spec():
    return pl.BlockSpec((BM, D), lambda i: (i, 0))


def _packed_spec():
    return pl.BlockSpec((BM, D2), lambda i: (i, 0))


def _full_spec(shape):
    return pl.BlockSpec(shape, lambda i: tuple(0 for _ in shape))


def _mm_relu(x, w, b):
    return pl.pallas_call(
        _mm_relu_body,
        grid=(GRID,),
        in_specs=[_row_spec(), _full_spec((D, D)), _full_spec((1, D))],
        out_specs=_packed_spec(),
        out_shape=jax.ShapeDtypeStruct((N, D2), jnp.int32),
    )(x, w, b)


def _lin(x, w, b):
    return pl.pallas_call(
        _lin_body,
        grid=(GRID,),
        in_specs=[_row_spec(), _full_spec((D, D)), _full_spec((1, D))],
        out_specs=_row_spec(),
        out_shape=jax.ShapeDtypeStruct((N, D), jnp.float32),
    )(x, w, b)


def _fc0(ft, agg, wb):
    return pl.pallas_call(
        _fc0_body,
        grid=(GRID,),
        in_specs=[_row_spec(), _packed_spec(), _full_spec((D, D))],
        out_specs=[_row_spec(), _full_spec((1, D)), _full_spec((1, D))],
        out_shape=[
            jax.ShapeDtypeStruct((N, D), jnp.float32),
            jax.ShapeDtypeStruct((1, D), jnp.float32),
            jax.ShapeDtypeStruct((1, D), jnp.float32),
        ],
    )(ft, agg, wb)


def _bn_norm_pool(z, s, ss, gamma, beta, wp, bp):
    return pl.pallas_call(
        _bn_body,
        grid=(GRID,),
        in_specs=[_row_spec(), _full_spec((1, D)), _full_spec((1, D)),
                  _full_spec((1, D)), _full_spec((1, D)),
                  _full_spec((D, D)), _full_spec((1, D))],
        out_specs=[_row_spec(), _packed_spec()],
        out_shape=[
            jax.ShapeDtypeStruct((N, D), jnp.float32),
            jax.ShapeDtypeStruct((N, D2), jnp.int32),
        ],
    )(z, s, ss, gamma, beta, wp, bp)


def _fc1(g, agg, wb):
    return pl.pallas_call(
        _fc1_body,
        grid=(GRID,),
        in_specs=[_row_spec(), _packed_spec(), _full_spec((D, D))],
        out_specs=_row_spec(),
        out_shape=jax.ShapeDtypeStruct((N, D), jnp.float32),
    )(g, agg, wb)


@jax.jit
def _run(features, neigh_idx, W_pool0, b_pool0, W_fc0, b_fc0, bn_gamma,
         bn_beta, W_pool1, b_pool1, W_fc1, b_fc1):
    idx_flat = jnp.pad(neigh_idx.astype(jnp.int32).reshape(-1),
                       (0, (CAP - N) * S))

    b_pool0 = b_pool0.reshape(1, D)
    b_fc0 = b_fc0.reshape(1, D)
    b_pool1 = b_pool1.reshape(1, D)
    b_fc1 = b_fc1.reshape(1, D)
    gamma = bn_gamma.reshape(1, D)
    beta = bn_beta.reshape(1, D)
    wa0, wb0 = W_fc0[:D], W_fc0[D:]
    wa1, wb1 = W_fc1[:D], W_fc1[D:]

    h0 = _mm_relu(features, W_pool0, b_pool0)
    agg0 = _sc_gather_max(h0, idx_flat)
    ft = _lin(features, wa0, b_fc0)
    z, s, ss = _fc0(ft, agg0, wb0)
    out1, h1 = _bn_norm_pool(z, s, ss, gamma, beta, W_pool1, b_pool1)
    agg1 = _sc_gather_max(h1, idx_flat)
    g = _lin(out1, wa1, b_fc1)
    return _fc1(g, agg1, wb1)


def kernel(features, neigh_idx, W_pool0, b_pool0, W_fc0, b_fc0, bn_gamma,
           bn_beta, W_pool1, b_pool1, W_fc1, b_fc1):
    return _run(features, neigh_idx, W_pool0, b_pool0, W_fc0, b_fc0,
                bn_gamma, bn_beta, W_pool1, b_pool1, W_fc1, b_fc1)


# speedup vs baseline: 1.0163x; 1.0163x over previous
import functools

import jax
import jax.numpy as jnp
from jax import lax
from jax.experimental import pallas as pl
from jax.experimental.pallas import tpu as pltpu
from jax.experimental.pallas import tpu_sc as plsc

N = 50000
D = 256
D2 = D // 2
S = 5
L = 16

NC = 2
NS = 16
NW = NC * NS

NPW0 = 1600
NPW1 = 1600
CAP = NW // 2 * (NPW0 + NPW1)
CHUNK = 40

BM = 5000
GRID = N // BM


def _sc_gather_max_body(table_hbm, idx_hbm, out_hbm, idx_v,
                        rows0, rows1, out0, out1,
                        gsem0, gsem1, osem0, osem1):
    c = lax.axis_index("c")
    s_ax = lax.axis_index("s")
    node_base = jnp.where(c == 0, s_ax * NPW0, NS * NPW0 + s_ax * NPW1)
    cap = jnp.where(c == 0, NPW0, NPW1)
    cnt = jnp.clip(N - node_base, 0, cap)
    nchunk = cnt // CHUNK
    npair = nchunk // 2

    @pl.when(c == 0)
    def _():
        pltpu.sync_copy(idx_hbm.at[pl.ds(node_base * S, NPW0 * S)],
                        idx_v.at[pl.ds(0, NPW0 * S)])

    @pl.when(c != 0)
    def _():
        pltpu.sync_copy(idx_hbm.at[pl.ds(node_base * S, NPW1 * S)],
                        idx_v.at[pl.ds(0, NPW1 * S)])

    rows = (rows0, rows1)
    outs = (out0, out1)
    gsems = (gsem0, gsem1)
    osems = (osem0, osem1)

    H0 = 96
    H1 = CHUNK * S - H0

    def gather_start(ci, b):
        base = ci * (CHUNK * S)
        pltpu.make_async_copy(
            table_hbm.at[idx_v.at[pl.ds(base, H0)]],
            rows[b].at[pl.ds(0, H0)], gsems[b]).start()
        pltpu.make_async_copy(
            table_hbm.at[idx_v.at[pl.ds(base + H0, H1)]],
            rows[b].at[pl.ds(H0, H1)], gsems[b]).start()

    def gather_wait(ci, b):
        pltpu.make_async_copy(
            table_hbm.at[idx_v.at[pl.ds(ci * (CHUNK * S), CHUNK * S)]],
            rows[b], gsems[b]).wait()

    def writeback(ci, b):
        return pltpu.make_async_copy(
            outs[b], out_hbm.at[pl.ds(node_base + ci * CHUNK, CHUNK)],
            osems[b])

    @pl.when(npair > 0)
    def _():
        gather_start(0, 0)

    def pair_body(pi, carry):
        for b in range(2):
            ci = pi * 2 + b
            nb = 1 - b

            @pl.when(ci + 1 < nchunk)
            def _():
                gather_start(ci + 1, nb)

            gather_wait(ci, b)

            @pl.when(ci >= 2)
            def _():
                writeback(ci - 2, b).wait()

            @plsc.parallel_loop(0, CHUNK, step=1, unroll=2)
            def node_body(j):
                r = j * S
                for k in range(D2 // L):
                    sl = pl.ds(k * L, L)
                    w = rows[b][r, sl]
                    lo = w & 0xFFFF
                    hi = lax.shift_right_logical(w, 16)
                    for s in range(1, S):
                        w = rows[b][r + s, sl]
                        lo = jnp.maximum(lo, w & 0xFFFF)
                        hi = jnp.maximum(hi, lax.shift_right_logical(w, 16))
                    outs[b][j, sl] = (hi << 16) | lo
            writeback(ci, b).start()
        return carry

    lax.fori_loop(0, npair, pair_body, 0, unroll=False)

    @pl.when(npair > 0)
    def _():
        writeback(nchunk - 2, 0).wait()
        writeback(nchunk - 1, 1).wait()


def _sc_gather_max(table, idx_flat):
    mesh = plsc.VectorSubcoreMesh(core_axis_name="c", subcore_axis_name="s")
    return pl.kernel(
        _sc_gather_max_body,
        out_type=jax.ShapeDtypeStruct((N, D2), jnp.int32),
        mesh=mesh,
        scratch_types=[
            pltpu.VMEM((NPW0 * S,), jnp.int32),
            pltpu.VMEM((CHUNK * S, D2), jnp.int32),
            pltpu.VMEM((CHUNK * S, D2), jnp.int32),
            pltpu.VMEM((CHUNK, D2), jnp.int32),
            pltpu.VMEM((CHUNK, D2), jnp.int32),
            pltpu.SemaphoreType.DMA,
            pltpu.SemaphoreType.DMA,
            pltpu.SemaphoreType.DMA,
            pltpu.SemaphoreType.DMA,
        ],
        name="sc_gather_max",
    )(table, idx_flat)


def _pack_bf16(x):
    u = jax.lax.bitcast_convert_type(x[:, :D2], jnp.uint32)
    v = jax.lax.bitcast_convert_type(x[:, D2:], jnp.uint32)
    lo = (u + 0x7FFF + ((u >> 16) & 1)) >> 16
    hi = (v + 0x7FFF + ((v >> 16) & 1)) & jnp.uint32(0xFFFF0000)
    return jax.lax.bitcast_convert_type(lo | hi, jnp.int32)


def _unpack_bf16(p):
    u = jax.lax.bitcast_convert_type(p, jnp.uint32)
    lo = jax.lax.bitcast_convert_type(u << 16, jnp.float32)
    hi = jax.lax.bitcast_convert_type(u & jnp.uint32(0xFFFF0000), jnp.float32)
    return jnp.concatenate([lo, hi], axis=1)


def _mm_relu_body(x_ref, w_ref, b_ref, o_ref):
    acc = jnp.dot(x_ref[...], w_ref[...], preferred_element_type=jnp.float32)
    o_ref[...] = _pack_bf16(jnp.maximum(acc + b_ref[...], 0.0))


def _lin_body(x_ref, w_ref, b_ref, o_ref):
    acc = jnp.dot(x_ref[...], w_ref[...], preferred_element_type=jnp.float32)
    o_ref[...] = acc + b_ref[...]


def _fc0_body(ft_ref, agg_ref, wb_ref, z_ref, s_ref, ss_ref):
    z = ft_ref[...] + jnp.dot(_unpack_bf16(agg_ref[...]), wb_ref[...],
                              preferred_element_type=jnp.float32)
    z = jnp.maximum(z, 0.0)
    z_ref[...] = z

    @pl.when(pl.program_id(0) == 0)
    def _():
        s_ref[...] = jnp.zeros_like(s_ref)
        ss_ref[...] = jnp.zeros_like(ss_ref)

    s_ref[...] += jnp.sum(z, axis=0, keepdims=True)
    ss_ref[...] += jnp.sum(z * z, axis=0, keepdims=True)


def _bn_body(z_ref, s_ref, ss_ref, g_ref, be_ref, wp_ref, bp_ref,
             on_ref, h1_ref):
    mean = s_ref[...] / N
    var = ss_ref[...] / N - mean * mean
    y = (z_ref[...] - mean) / jnp.sqrt(var + 1e-5) * g_ref[...] + be_ref[...]
    nrm = jnp.sqrt(jnp.sum(y * y, axis=1, keepdims=True)) + 1e-6
    on = y / nrm
    on_ref[...] = on
    acc = jnp.dot(on, wp_ref[...], preferred_element_type=jnp.float32)
    h1_ref[...] = _pack_bf16(jnp.maximum(acc + bp_ref[...], 0.0))


def _fc1_body(g_ref, agg_ref, wb_ref, o_ref):
    o_ref[...] = g_ref[...] + jnp.dot(_unpack_bf16(agg_ref[...]), wb_ref[...],
                                      preferred_element_type=jnp.float32)


def _row_spec():
    return pl.BlockSpec((BM, D), lambda i: (i, 0))


def _packed_spec():
    return pl.BlockSpec((BM, D2), lambda i: (i, 0))


def _full_spec(shape):
    return pl.BlockSpec(shape, lambda i: tuple(0 for _ in shape))


def _mm_relu(x, w, b):
    return pl.pallas_call(
        _mm_relu_body,
        grid=(GRID,),
        in_specs=[_row_spec(), _full_spec((D, D)), _full_spec((1, D))],
        out_specs=_packed_spec(),
        out_shape=jax.ShapeDtypeStruct((N, D2), jnp.int32),
    )(x, w, b)


def _lin(x, w, b):
    return pl.pallas_call(
        _lin_body,
        grid=(GRID,),
        in_specs=[_row_spec(), _full_spec((D, D)), _full_spec((1, D))],
        out_specs=_row_spec(),
        out_shape=jax.ShapeDtypeStruct((N, D), jnp.float32),
    )(x, w, b)


def _fc0(ft, agg, wb):
    return pl.pallas_call(
        _fc0_body,
        grid=(GRID,),
        in_specs=[_row_spec(), _packed_spec(), _full_spec((D, D))],
        out_specs=[_row_spec(), _full_spec((1, D)), _full_spec((1, D))],
        out_shape=[
            jax.ShapeDtypeStruct((N, D), jnp.float32),
            jax.ShapeDtypeStruct((1, D), jnp.float32),
            jax.ShapeDtypeStruct((1, D), jnp.float32),
        ],
    )(ft, agg, wb)


def _bn_norm_pool(z, s, ss, gamma, beta, wp, bp):
    return pl.pallas_call(
        _bn_body,
        grid=(GRID,),
        in_specs=[_row_spec(), _full_spec((1, D)), _full_spec((1, D)),
                  _full_spec((1, D)), _full_spec((1, D)),
                  _full_spec((D, D)), _full_spec((1, D))],
        out_specs=[_row_spec(), _packed_spec()],
        out_shape=[
            jax.ShapeDtypeStruct((N, D), jnp.float32),
            jax.ShapeDtypeStruct((N, D2), jnp.int32),
        ],
    )(z, s, ss, gamma, beta, wp, bp)


def _fc1(g, agg, wb):
    return pl.pallas_call(
        _fc1_body,
        grid=(GRID,),
        in_specs=[_row_spec(), _packed_spec(), _full_spec((D, D))],
        out_specs=_row_spec(),
        out_shape=jax.ShapeDtypeStruct((N, D), jnp.float32),
    )(g, agg, wb)


@jax.jit
def _run(features, neigh_idx, W_pool0, b_pool0, W_fc0, b_fc0, bn_gamma,
         bn_beta, W_pool1, b_pool1, W_fc1, b_fc1):
    idx_flat = jnp.pad(neigh_idx.astype(jnp.int32).reshape(-1),
                       (0, (CAP - N) * S))

    b_pool0 = b_pool0.reshape(1, D)
    b_fc0 = b_fc0.reshape(1, D)
    b_pool1 = b_pool1.reshape(1, D)
    b_fc1 = b_fc1.reshape(1, D)
    gamma = bn_gamma.reshape(1, D)
    beta = bn_beta.reshape(1, D)
    wa0, wb0 = W_fc0[:D], W_fc0[D:]
    wa1, wb1 = W_fc1[:D], W_fc1[D:]

    h0 = _mm_relu(features, W_pool0, b_pool0)
    agg0 = _sc_gather_max(h0, idx_flat)
    ft = _lin(features, wa0, b_fc0)
    z, s, ss = _fc0(ft, agg0, wb0)
    out1, h1 = _bn_norm_pool(z, s, ss, gamma, beta, W_pool1, b_pool1)
    agg1 = _sc_gather_max(h1, idx_flat)
    g = _lin(out1, wa1, b_fc1)
    return _fc1(g, agg1, wb1)


def kernel(features, neigh_idx, W_pool0, b_pool0, W_fc0, b_fc0, bn_gamma,
           bn_beta, W_pool1, b_pool1, W_fc1, b_fc1):
    return _run(features, neigh_idx, W_pool0, b_pool0, W_fc0, b_fc0,
                bn_gamma, bn_beta, W_pool1, b_pool1, W_fc1, b_fc1)
